# baseline (speedup 1.0000x reference)
import functools

import jax
import jax.numpy as jnp
from jax import lax
from jax.experimental import pallas as pl
from jax.experimental.pallas import tpu as pltpu
from jax.experimental.pallas import tpu_sc as plsc

VOCAB = 1000000
EMBED = 16
BATCH = 16384
HIST = 200

NC = 2
NS = 16
L = 16
NW = NC * NS
B_PER_W = BATCH // NW
NACC = B_PER_W // L
SLAB_L = 4
SLAB = SLAB_L * B_PER_W
N_IT = HIST // SLAB_L

_TV_COLS = 32768


def _tv_body(tbl_ref, v_ref, out_ref):
  out_ref[...] = jnp.sum(tbl_ref[...] * v_ref[...], axis=0)


def _table_matvec(table_t, v):
  return pl.pallas_call(
      _tv_body,
      grid=(pl.cdiv(VOCAB, _TV_COLS),),
      in_specs=[
          pl.BlockSpec((EMBED, _TV_COLS), lambda i: (0, i)),
          pl.BlockSpec((EMBED, 1), lambda i: (0, 0)),
      ],
      out_specs=pl.BlockSpec((_TV_COLS,), lambda i: (i,)),
      out_shape=jax.ShapeDtypeStruct((VOCAB,), jnp.float32),
  )(table_t, v)


def _make_sc_kernel():
  mesh = plsc.VectorSubcoreMesh(core_axis_name="c", subcore_axis_name="s")

  @functools.partial(
      pl.kernel,
      mesh=mesh,
      compiler_params=pltpu.CompilerParams(use_tc_tiling_on_sc=False),
      out_type=jax.ShapeDtypeStruct((BATCH,), jnp.float32),
      scratch_types=[
          pltpu.VMEM_SHARED((VOCAB,), jnp.float32),
          pltpu.VMEM((SLAB,), jnp.int32),
          pltpu.VMEM((SLAB,), jnp.int32),
          pltpu.VMEM((SLAB,), jnp.float32),
          pltpu.VMEM((SLAB,), jnp.float32),
          pltpu.VMEM((L,), jnp.float32),
          pltpu.VMEM((B_PER_W,), jnp.float32),
          pltpu.SemaphoreType.DMA,
          pltpu.SemaphoreType.DMA,
          pltpu.SemaphoreType.DMA,
          pltpu.SemaphoreType.DMA,
      ],
  )
  def sc_pool(xtflat, tvals, cc, out, shared_t, xs0, xs1, val0, val1, cc_v,
              out_v, sx0, sx1, sg0, sg1):
    sid = lax.axis_index("s")
    wid = sid * NC + lax.axis_index("c")
    col0 = wid * B_PER_W
    xsb = (xs0, xs1)
    sxb = (sx0, sx1)
    valb = (val0, val1)
    sgb = (sg0, sg1)
    pltpu.sync_copy(cc, cc_v)
    ccvec = cc_v[...]
    inv = jnp.float32(1.0 / HIST)

    def stage(it, b):
      for q in range(SLAB_L):
        pltpu.async_copy(
            xtflat.at[pl.ds((it * SLAB_L + q) * BATCH + col0, B_PER_W)],
            xsb[b].at[pl.ds(q * B_PER_W, B_PER_W)], sxb[b])

    def stage_wait(b):
      pltpu.make_async_copy(xtflat.at[pl.ds(0, SLAB)], xsb[b],
                            sxb[b]).wait()

    def g_start(b):
      pltpu.async_copy(shared_t.at[xsb[b]], valb[b], sgb[b])

    def g_wait(b):
      pltpu.make_async_copy(shared_t.at[xsb[b]], valb[b], sgb[b]).wait()

    stage(0, 0)
    stage(1, 1)

    @pl.when(sid == 0)
    def _():
      pltpu.sync_copy(tvals, shared_t)

    plsc.subcore_barrier()
    stage_wait(0)
    g_start(0)

    def pair_body(lp, accs):
      for half in (0, 1):
        it = lp * 2 + half
        b = half
        nb = 1 - half

        @pl.when(it + 1 < N_IT)
        def _():
          stage_wait(nb)
          g_start(nb)

        g_wait(b)

        @pl.when(it + 2 < N_IT)
        def _():
          stage(it + 2, b)

        vv = valb[b]
        for q in range(SLAB_L):
          accs = tuple(accs[j] + vv[pl.ds((q * NACC + j) * L, L)]
                       for j in range(NACC))
      return accs

    zero = jnp.zeros((L,), jnp.float32)
    accs = lax.fori_loop(0, N_IT // 2, pair_body, (zero,) * NACC)
    for j in range(NACC):
      z = accs[j] * inv + ccvec
      out_v[pl.ds(j * L, L)] = 1.0 / (1.0 + jnp.exp(-z))
    pltpu.sync_copy(out_v, out.at[pl.ds(col0, B_PER_W)])

  return sc_pool


_SC_POOL = _make_sc_kernel()


def kernel(x, table, W1, b1, W2, b2):
  v = (W2 @ W1).reshape(EMBED, 1).astype(jnp.float32)
  c = (W2 @ b1 + b2).reshape(())
  cc = jnp.full((L,), c, jnp.float32)
  tvals = _table_matvec(table.T, v)
  out = _SC_POOL(x.T.reshape(-1), tvals, cc)
  return out.reshape(BATCH, 1)


# speedup vs baseline: 1.0603x; 1.0603x over previous
import functools

import jax
import jax.numpy as jnp
from jax import lax
from jax.experimental import pallas as pl
from jax.experimental.pallas import tpu as pltpu
from jax.experimental.pallas import tpu_sc as plsc

VOCAB = 1000000
EMBED = 16
BATCH = 16384
HIST = 200

NC = 2
NS = 16
L = 16
NW = NC * NS
B_PER_W = BATCH // NW
NACC = B_PER_W // L
SLAB_L = 4
SLAB = SLAB_L * B_PER_W
N_IT = HIST // SLAB_L

_TV_COLS = 65536


def _tv_body(tbl_ref, v_ref, out_ref):
  out_ref[...] = jnp.sum(tbl_ref[...] * v_ref[...], axis=0)


def _table_matvec(table_t, v):
  return pl.pallas_call(
      _tv_body,
      grid=(pl.cdiv(VOCAB, _TV_COLS),),
      in_specs=[
          pl.BlockSpec((EMBED, _TV_COLS), lambda i: (0, i)),
          pl.BlockSpec((EMBED, 1), lambda i: (0, 0)),
      ],
      out_specs=pl.BlockSpec((_TV_COLS,), lambda i: (i,)),
      out_shape=jax.ShapeDtypeStruct((VOCAB,), jnp.float32),
  )(table_t, v)


def _make_sc_kernel():
  mesh = plsc.VectorSubcoreMesh(core_axis_name="c", subcore_axis_name="s")

  @functools.partial(
      pl.kernel,
      mesh=mesh,
      compiler_params=pltpu.CompilerParams(use_tc_tiling_on_sc=False),
      out_type=jax.ShapeDtypeStruct((BATCH,), jnp.float32),
      scratch_types=[
          pltpu.VMEM_SHARED((VOCAB,), jnp.float32),
          pltpu.VMEM((SLAB,), jnp.int32),
          pltpu.VMEM((SLAB,), jnp.int32),
          pltpu.VMEM((SLAB,), jnp.float32),
          pltpu.VMEM((SLAB,), jnp.float32),
          pltpu.VMEM((L,), jnp.float32),
          pltpu.VMEM((B_PER_W,), jnp.float32),
          pltpu.SemaphoreType.DMA,
          pltpu.SemaphoreType.DMA,
          pltpu.SemaphoreType.DMA,
          pltpu.SemaphoreType.DMA,
      ],
  )
  def sc_pool(xtflat, tvals, cc, out, shared_t, xs0, xs1, val0, val1, cc_v,
              out_v, sx0, sx1, sg0, sg1):
    sid = lax.axis_index("s")
    wid = sid * NC + lax.axis_index("c")
    col0 = wid * B_PER_W
    xsb = (xs0, xs1)
    sxb = (sx0, sx1)
    valb = (val0, val1)
    sgb = (sg0, sg1)
    pltpu.sync_copy(cc, cc_v)
    ccvec = cc_v[...]
    inv = jnp.float32(1.0 / HIST)

    def stage(it, b):
      for q in range(SLAB_L):
        pltpu.async_copy(
            xtflat.at[pl.ds((it * SLAB_L + q) * BATCH + col0, B_PER_W)],
            xsb[b].at[pl.ds(q * B_PER_W, B_PER_W)], sxb[b])

    def stage_wait(b):
      pltpu.make_async_copy(xtflat.at[pl.ds(0, SLAB)], xsb[b],
                            sxb[b]).wait()

    def g_start(b):
      pltpu.async_copy(shared_t.at[xsb[b]], valb[b], sgb[b])

    def g_wait(b):
      pltpu.make_async_copy(shared_t.at[xsb[b]], valb[b], sgb[b]).wait()

    stage(0, 0)
    stage(1, 1)

    @pl.when(sid == 0)
    def _():
      pltpu.sync_copy(tvals, shared_t)

    plsc.subcore_barrier()
    stage_wait(0)
    g_start(0)

    def pair_body(lp, accs):
      for half in (0, 1):
        it = lp * 2 + half
        b = half
        nb = 1 - half

        @pl.when(it + 1 < N_IT)
        def _():
          stage_wait(nb)
          g_start(nb)

        g_wait(b)

        @pl.when(it + 2 < N_IT)
        def _():
          stage(it + 2, b)

        vv = valb[b]
        for q in range(SLAB_L):
          accs = tuple(accs[j] + vv[pl.ds((q * NACC + j) * L, L)]
                       for j in range(NACC))
      return accs

    zero = jnp.zeros((L,), jnp.float32)
    accs = lax.fori_loop(0, N_IT // 2, pair_body, (zero,) * NACC)
    for j in range(NACC):
      z = accs[j] * inv + ccvec
      out_v[pl.ds(j * L, L)] = 1.0 / (1.0 + jnp.exp(-z))
    pltpu.sync_copy(out_v, out.at[pl.ds(col0, B_PER_W)])

  return sc_pool


_SC_POOL = _make_sc_kernel()


def kernel(x, table, W1, b1, W2, b2):
  v = (W2 @ W1).reshape(EMBED, 1).astype(jnp.float32)
  c = (W2 @ b1 + b2).reshape(())
  cc = jnp.full((L,), c, jnp.float32)
  tvals = _table_matvec(table.T, v)
  out = _SC_POOL(x.T.reshape(-1), tvals, cc)
  return out.reshape(BATCH, 1)


# speedup vs baseline: 1.0982x; 1.0357x over previous
import functools

import jax
import jax.numpy as jnp
from jax import lax
from jax.experimental import pallas as pl
from jax.experimental.pallas import tpu as pltpu
from jax.experimental.pallas import tpu_sc as plsc

VOCAB = 1000000
EMBED = 16
BATCH = 16384
HIST = 200

NC = 2
NS = 16
L = 16
NW = NC * NS
B_PER_W = BATCH // NW
NACC = B_PER_W // L
SLAB_L = 4
SLAB = SLAB_L * B_PER_W
N_IT = HIST // SLAB_L

_TV_COLS = 65536


def _tv_body(tbl_ref, v_ref, out_ref):
  z = jax.lax.dot_general(v_ref[...], tbl_ref[...], (((1,), (0,)), ((), ())),
                          preferred_element_type=jnp.float32)
  out_ref[...] = z[0]


def _table_matvec(table_t, v):
  return pl.pallas_call(
      _tv_body,
      grid=(pl.cdiv(VOCAB, _TV_COLS),),
      in_specs=[
          pl.BlockSpec((EMBED, _TV_COLS), lambda i: (0, i)),
          pl.BlockSpec((1, EMBED), lambda i: (0, 0)),
      ],
      out_specs=pl.BlockSpec((_TV_COLS,), lambda i: (i,)),
      out_shape=jax.ShapeDtypeStruct((VOCAB,), jnp.float32),
  )(table_t, v)


def _make_sc_kernel():
  mesh = plsc.VectorSubcoreMesh(core_axis_name="c", subcore_axis_name="s")

  @functools.partial(
      pl.kernel,
      mesh=mesh,
      compiler_params=pltpu.CompilerParams(use_tc_tiling_on_sc=False),
      out_type=jax.ShapeDtypeStruct((BATCH,), jnp.float32),
      scratch_types=[
          pltpu.VMEM_SHARED((VOCAB,), jnp.float32),
          pltpu.VMEM((SLAB,), jnp.int32),
          pltpu.VMEM((SLAB,), jnp.int32),
          pltpu.VMEM((SLAB,), jnp.float32),
          pltpu.VMEM((SLAB,), jnp.float32),
          pltpu.VMEM((L,), jnp.float32),
          pltpu.VMEM((B_PER_W,), jnp.float32),
          pltpu.SemaphoreType.DMA,
          pltpu.SemaphoreType.DMA,
          pltpu.SemaphoreType.DMA,
          pltpu.SemaphoreType.DMA,
      ],
  )
  def sc_pool(xtflat, tvals, cc, out, shared_t, xs0, xs1, val0, val1, cc_v,
              out_v, sx0, sx1, sg0, sg1):
    sid = lax.axis_index("s")
    wid = sid * NC + lax.axis_index("c")
    col0 = wid * B_PER_W
    xsb = (xs0, xs1)
    sxb = (sx0, sx1)
    valb = (val0, val1)
    sgb = (sg0, sg1)
    pltpu.sync_copy(cc, cc_v)
    ccvec = cc_v[...]
    inv = jnp.float32(1.0 / HIST)

    def stage(it, b):
      for q in range(SLAB_L):
        pltpu.async_copy(
            xtflat.at[pl.ds((it * SLAB_L + q) * BATCH + col0, B_PER_W)],
            xsb[b].at[pl.ds(q * B_PER_W, B_PER_W)], sxb[b])

    def stage_wait(b):
      pltpu.make_async_copy(xtflat.at[pl.ds(0, SLAB)], xsb[b],
                            sxb[b]).wait()

    def g_start(b):
      pltpu.async_copy(shared_t.at[xsb[b]], valb[b], sgb[b])

    def g_wait(b):
      pltpu.make_async_copy(shared_t.at[xsb[b]], valb[b], sgb[b]).wait()

    stage(0, 0)
    stage(1, 1)

    @pl.when(sid == 0)
    def _():
      pltpu.sync_copy(tvals, shared_t)

    plsc.subcore_barrier()
    stage_wait(0)
    g_start(0)

    def pair_body(lp, accs):
      for half in (0, 1):
        it = lp * 2 + half
        b = half
        nb = 1 - half

        @pl.when(it + 1 < N_IT)
        def _():
          stage_wait(nb)
          g_start(nb)

        g_wait(b)

        @pl.when(it + 2 < N_IT)
        def _():
          stage(it + 2, b)

        vv = valb[b]
        for q in range(SLAB_L):
          accs = tuple(accs[j] + vv[pl.ds((q * NACC + j) * L, L)]
                       for j in range(NACC))
      return accs

    zero = jnp.zeros((L,), jnp.float32)
    accs = lax.fori_loop(0, N_IT // 2, pair_body, (zero,) * NACC)
    for j in range(NACC):
      z = accs[j] * inv + ccvec
      out_v[pl.ds(j * L, L)] = 1.0 / (1.0 + jnp.exp(-z))
    pltpu.sync_copy(out_v, out.at[pl.ds(col0, B_PER_W)])

  return sc_pool


_SC_POOL = _make_sc_kernel()


def kernel(x, table, W1, b1, W2, b2):
  v = (W2 @ W1).reshape(1, EMBED).astype(jnp.float32)
  c = (W2 @ b1 + b2).reshape(())
  cc = jnp.full((L,), c, jnp.float32)
  tvals = _table_matvec(table.T, v)
  out = _SC_POOL(x.T.reshape(-1), tvals, cc)
  return out.reshape(BATCH, 1)
